# baseline (speedup 1.0000x reference)
import functools

import jax
import jax.numpy as jnp
from jax import lax
from jax.experimental import pallas as pl
from jax.experimental.pallas import tpu as pltpu
from jax.experimental.pallas import tpu_sc as plsc

_NC = 2
_NS = 16
_NW = _NC * _NS
_LANES = 16


def _sc_gather_mul(F0, F1, idx0, idx1):
    B = idx0.shape[0]
    K = F0.shape[1]
    b_per_w = B // _NW
    mesh = plsc.VectorSubcoreMesh(core_axis_name="c", subcore_axis_name="s")

    @functools.partial(
        pl.kernel,
        mesh=mesh,
        out_type=jax.ShapeDtypeStruct((B, K), jnp.float32),
        scratch_types=[
            pltpu.VMEM((b_per_w,), jnp.int32),
            pltpu.VMEM((b_per_w,), jnp.int32),
            pltpu.VMEM((b_per_w, K), jnp.float32),
            pltpu.VMEM((b_per_w, K), jnp.float32),
            pltpu.SemaphoreType.DMA,
            pltpu.SemaphoreType.DMA,
        ],
    )
    def sc_kernel(idx0_hbm, idx1_hbm, f0_hbm, f1_hbm, out_hbm,
                  i0_v, i1_v, r0_v, r1_v, sem0, sem1):
        wid = lax.axis_index("s") * _NC + lax.axis_index("c")
        base = wid * b_per_w
        pltpu.sync_copy(idx0_hbm.at[pl.ds(base, b_per_w)], i0_v)
        pltpu.sync_copy(idx1_hbm.at[pl.ds(base, b_per_w)], i1_v)
        c0 = pltpu.async_copy(f0_hbm.at[i0_v], r0_v, sem0)
        c1 = pltpu.async_copy(f1_hbm.at[i1_v], r1_v, sem1)
        c0.wait()
        c1.wait()

        def row_body(r, carry):
            for j in range(K // _LANES):
                sl = pl.ds(j * _LANES, _LANES)
                r0_v[r, sl] = r0_v[r, sl] * r1_v[r, sl]
            return carry

        lax.fori_loop(0, b_per_w, row_body, 0, unroll=2)
        pltpu.sync_copy(r0_v, out_hbm.at[pl.ds(base, b_per_w)])

    return sc_kernel(idx0, idx1, F0, F1)


def _tc_matmul(prod, F2):
    B, K = prod.shape
    N = F2.shape[0]
    BLK = 512

    def mm_body(p_ref, f2_ref, o_ref):
        o_ref[...] = lax.dot_general(
            p_ref[...], f2_ref[...],
            (((1,), (1,)), ((), ())),
            preferred_element_type=jnp.float32,
        )

    return pl.pallas_call(
        mm_body,
        grid=(B // BLK,),
        in_specs=[
            pl.BlockSpec((BLK, K), lambda i: (i, 0)),
            pl.BlockSpec((N, K), lambda i: (0, 0)),
        ],
        out_specs=pl.BlockSpec((BLK, N), lambda i: (i, 0)),
        out_shape=jax.ShapeDtypeStruct((B, N), jnp.float32),
    )(prod, F2)


def kernel(indices, F0, F1, F2, log_sigma):
    idx0 = indices[:, 0].astype(jnp.int32)
    idx1 = indices[:, 1].astype(jnp.int32)
    prod = _sc_gather_mul(F0, F1, idx0, idx1)
    res = _tc_matmul(prod, F2)
    return (res, log_sigma)
